# baseline (speedup 1.0000x reference)
import functools

import jax
import jax.numpy as jnp
from jax.experimental import pallas as pl
from jax.experimental.pallas import tpu as pltpu


def _fused_body(x_ref, wck_ref, bc_ref, wl_ref, bl_ref, out_ref,
                *, TB, P, pool_out):
    D = x_ref.shape[2]
    F = bc_ref.shape[1]
    x2 = x_ref[0]

    top = jnp.dot(x2[0:TB], wck_ref[D:2 * D, 0:F],
                  preferred_element_type=jnp.float32)

    T = pool_out * P
    half = (T // 2 + P - 1) // P * P
    chunk_t = [(0, half), (half, T)]
    bc = bc_ref[...]
    acc = bl_ref[...]
    for t0, t1 in chunk_t:
        lo = max(t0 - 1, 0)
        nblk = t1 + 2 - lo
        lhs = jnp.concatenate(
            [x2[lo * TB:(lo + nblk) * TB],
             x2[(lo + 1) * TB:(lo + 1 + nblk) * TB],
             x2[(lo + 2) * TB:(lo + 2 + nblk) * TB]], axis=1)
        z = jnp.dot(lhs, wck_ref[...], preferred_element_type=jnp.float32)
        gs = []
        for j in range(t0 // P, t1 // P):
            g = None
            for p in range(P):
                t = j * P + p
                s = z[(t + 1 - lo) * TB:(t + 2 - lo) * TB, F:2 * F]
                if t == 0:
                    s = s + top
                else:
                    s = s + z[(t - 1 - lo) * TB:(t - lo) * TB, 0:F]
                g = s if g is None else jnp.maximum(g, s)
            g = jnp.maximum(g + bc, 0.0)
            gs.append(g.astype(jnp.bfloat16))
        gcat = jnp.concatenate(gs, axis=1)
        acc = acc + jnp.dot(
            gcat, wl_ref[(t0 // P) * F:(t1 // P) * F, :],
            preferred_element_type=jnp.float32)
    out_ref[...] = acc.astype(out_ref.dtype)


def kernel(x, conv_w, conv_b, lin_w, lin_b):
    B, c_in, L, D = x.shape
    F = conv_w.shape[0]
    C = lin_w.shape[0]
    K = 5
    P = 4
    pool_out = (L - K + 1) // P
    H = L
    TB = 128
    num_tiles = B // TB
    f32 = jnp.float32


    wc = jnp.transpose(conv_w[:, 0], (1, 2, 0)).astype(jnp.bfloat16)
    zer = jnp.zeros((D, F), jnp.bfloat16)
    wck = jnp.concatenate([
        jnp.concatenate([wc[0], wc[2]], axis=1),
        jnp.concatenate([wc[1], wc[3]], axis=1),
        jnp.concatenate([zer, wc[4]], axis=1),
    ], axis=0)
    bc = conv_b.reshape(1, F).astype(f32)

    wl2 = lin_w.reshape(C, F, pool_out)
    wl2 = jnp.transpose(wl2, (2, 1, 0)).reshape(pool_out * F, C)
    wl2 = wl2.astype(jnp.bfloat16)
    bl = lin_b.reshape(1, C).astype(f32)

    body = functools.partial(_fused_body, TB=TB, P=P, pool_out=pool_out)
    xb = x.reshape(num_tiles, TB, H, D).transpose(0, 2, 1, 3)
    xb = xb.astype(jnp.bfloat16).reshape(num_tiles, H * TB, D)
    out = pl.pallas_call(
        body,
        out_shape=jax.ShapeDtypeStruct((B, C), f32),
        grid=(num_tiles,),
        in_specs=[
            pl.BlockSpec((1, H * TB, D), lambda i: (i, 0, 0)),
            pl.BlockSpec((3 * D, 2 * F), lambda i: (0, 0)),
            pl.BlockSpec((1, F), lambda i: (0, 0)),
            pl.BlockSpec((pool_out * F, C), lambda i: (0, 0)),
            pl.BlockSpec((1, C), lambda i: (0, 0)),
        ],
        out_specs=pl.BlockSpec((TB, C), lambda i: (i, 0)),
        compiler_params=pltpu.CompilerParams(
            dimension_semantics=("parallel",),
            vmem_limit_bytes=60 * 1024 * 1024),
    )(xb, wck, bc, wl2, bl)
    return out


# speedup vs baseline: 1.2825x; 1.2825x over previous
import functools

import jax
import jax.numpy as jnp
from jax.experimental import pallas as pl
from jax.experimental.pallas import tpu as pltpu


def _fused_body(x_ref, wck_ref, bc_ref, wl_ref, bl_ref, out_ref,
                *, TB, P, pool_out):
    D = x_ref.shape[2]
    F = bc_ref.shape[1]
    x2 = x_ref[0]

    top = jnp.dot(x2[0:TB], wck_ref[D:2 * D, 0:F],
                  preferred_element_type=jnp.float32)

    T = pool_out * P
    half = (T // 2 + P - 1) // P * P
    chunk_t = [(0, half), (half, T)]
    bc = bc_ref[...]
    gs = []
    for t0, t1 in chunk_t:
        lo = max(t0 - 1, 0)
        nblk = t1 + 2 - lo
        lhs = jnp.concatenate(
            [x2[lo * TB:(lo + nblk) * TB],
             x2[(lo + 1) * TB:(lo + 1 + nblk) * TB],
             x2[(lo + 2) * TB:(lo + 2 + nblk) * TB]], axis=1)
        z = jnp.dot(lhs, wck_ref[...], preferred_element_type=jnp.float32)
        for j in range(t0 // P, t1 // P):
            g = None
            for p in range(P):
                t = j * P + p
                s = z[(t + 1 - lo) * TB:(t + 2 - lo) * TB, F:2 * F]
                if t == 0:
                    s = s + top
                else:
                    s = s + z[(t - 1 - lo) * TB:(t - lo) * TB, 0:F]
                g = s if g is None else jnp.maximum(g, s)
            g = jnp.maximum(g + bc, 0.0)
            gs.append(g.astype(jnp.bfloat16))
    gcat = jnp.concatenate(gs, axis=1)

    out_ref[...] = (jnp.dot(gcat, wl_ref[...],
                            preferred_element_type=jnp.float32)
                    + bl_ref[...]).astype(out_ref.dtype)


def kernel(x, conv_w, conv_b, lin_w, lin_b):
    B, c_in, L, D = x.shape
    F = conv_w.shape[0]
    C = lin_w.shape[0]
    K = 5
    P = 4
    pool_out = (L - K + 1) // P
    H = L
    TB = 128
    num_tiles = B // TB
    f32 = jnp.float32


    wc = jnp.transpose(conv_w[:, 0], (1, 2, 0)).astype(jnp.bfloat16)
    zer = jnp.zeros((D, F), jnp.bfloat16)
    wck = jnp.concatenate([
        jnp.concatenate([wc[0], wc[2]], axis=1),
        jnp.concatenate([wc[1], wc[3]], axis=1),
        jnp.concatenate([zer, wc[4]], axis=1),
    ], axis=0)
    bc = conv_b.reshape(1, F).astype(f32)

    wl2 = lin_w.reshape(C, F, pool_out)
    wl2 = jnp.transpose(wl2, (2, 1, 0)).reshape(pool_out * F, C)
    wl2 = wl2.astype(jnp.bfloat16)
    bl = lin_b.reshape(1, C).astype(f32)

    body = functools.partial(_fused_body, TB=TB, P=P, pool_out=pool_out)
    xb = x.reshape(num_tiles, TB, H, D).transpose(0, 2, 1, 3)
    xb = xb.astype(jnp.bfloat16).reshape(num_tiles, H * TB, D)
    out = pl.pallas_call(
        body,
        out_shape=jax.ShapeDtypeStruct((B, C), f32),
        grid=(num_tiles,),
        in_specs=[
            pl.BlockSpec((1, H * TB, D), lambda i: (i, 0, 0)),
            pl.BlockSpec((3 * D, 2 * F), lambda i: (0, 0)),
            pl.BlockSpec((1, F), lambda i: (0, 0)),
            pl.BlockSpec((pool_out * F, C), lambda i: (0, 0)),
            pl.BlockSpec((1, C), lambda i: (0, 0)),
        ],
        out_specs=pl.BlockSpec((TB, C), lambda i: (i, 0)),
        compiler_params=pltpu.CompilerParams(
            dimension_semantics=("parallel",),
            vmem_limit_bytes=60 * 1024 * 1024),
    )(xb, wck, bc, wl2, bl)
    return out


# speedup vs baseline: 1.2864x; 1.0030x over previous
import functools

import jax
import jax.numpy as jnp
from jax.experimental import pallas as pl
from jax.experimental.pallas import tpu as pltpu


def _fused_body(x_ref, wck_ref, bc_ref, wl_ref, bl_ref, out_ref,
                *, TB, P, pool_out):
    D = x_ref.shape[2]
    F = bc_ref.shape[1]
    x2 = x_ref[0]

    top = jnp.dot(x2[0:TB], wck_ref[D:2 * D, 0:F],
                  preferred_element_type=jnp.float32)

    T = pool_out * P
    chunk_t = [(0, T)]
    bc = bc_ref[...]
    gs = []
    for t0, t1 in chunk_t:
        lo = max(t0 - 1, 0)
        nblk = t1 + 2 - lo
        lhs = jnp.concatenate(
            [x2[lo * TB:(lo + nblk) * TB],
             x2[(lo + 1) * TB:(lo + 1 + nblk) * TB],
             x2[(lo + 2) * TB:(lo + 2 + nblk) * TB]], axis=1)
        z = jnp.dot(lhs, wck_ref[...], preferred_element_type=jnp.float32)
        for j in range(t0 // P, t1 // P):
            g = None
            for p in range(P):
                t = j * P + p
                s = z[(t + 1 - lo) * TB:(t + 2 - lo) * TB, F:2 * F]
                if t == 0:
                    s = s + top
                else:
                    s = s + z[(t - 1 - lo) * TB:(t - lo) * TB, 0:F]
                g = s if g is None else jnp.maximum(g, s)
            g = jnp.maximum(g + bc, 0.0)
            gs.append(g.astype(jnp.bfloat16))
    gcat = jnp.concatenate(gs, axis=1)

    out_ref[...] = (jnp.dot(gcat, wl_ref[...],
                            preferred_element_type=jnp.float32)
                    + bl_ref[...]).astype(out_ref.dtype)


def kernel(x, conv_w, conv_b, lin_w, lin_b):
    B, c_in, L, D = x.shape
    F = conv_w.shape[0]
    C = lin_w.shape[0]
    K = 5
    P = 4
    pool_out = (L - K + 1) // P
    H = L
    TB = 128
    num_tiles = B // TB
    f32 = jnp.float32


    wc = jnp.transpose(conv_w[:, 0], (1, 2, 0)).astype(jnp.bfloat16)
    zer = jnp.zeros((D, F), jnp.bfloat16)
    wck = jnp.concatenate([
        jnp.concatenate([wc[0], wc[2]], axis=1),
        jnp.concatenate([wc[1], wc[3]], axis=1),
        jnp.concatenate([zer, wc[4]], axis=1),
    ], axis=0)
    bc = conv_b.reshape(1, F).astype(f32)

    wl2 = lin_w.reshape(C, F, pool_out)
    wl2 = jnp.transpose(wl2, (2, 1, 0)).reshape(pool_out * F, C)
    wl2 = wl2.astype(jnp.bfloat16)
    bl = lin_b.reshape(1, C).astype(f32)

    body = functools.partial(_fused_body, TB=TB, P=P, pool_out=pool_out)
    xb = x.reshape(num_tiles, TB, H, D).transpose(0, 2, 1, 3)
    xb = xb.astype(jnp.bfloat16).reshape(num_tiles, H * TB, D)
    out = pl.pallas_call(
        body,
        out_shape=jax.ShapeDtypeStruct((B, C), f32),
        grid=(num_tiles,),
        in_specs=[
            pl.BlockSpec((1, H * TB, D), lambda i: (i, 0, 0)),
            pl.BlockSpec((3 * D, 2 * F), lambda i: (0, 0)),
            pl.BlockSpec((1, F), lambda i: (0, 0)),
            pl.BlockSpec((pool_out * F, C), lambda i: (0, 0)),
            pl.BlockSpec((1, C), lambda i: (0, 0)),
        ],
        out_specs=pl.BlockSpec((TB, C), lambda i: (i, 0)),
        compiler_params=pltpu.CompilerParams(
            dimension_semantics=("parallel",),
            vmem_limit_bytes=60 * 1024 * 1024),
    )(xb, wck, bc, wl2, bl)
    return out
